# baseline (speedup 1.0000x reference)
import functools

import jax
import jax.numpy as jnp
from jax import lax
from jax.experimental import pallas as pl
from jax.experimental.pallas import tpu as pltpu
from jax.experimental.pallas import tpu_sc as plsc

N = 16384
K = 8192
D = 256
BN = 2048
BK = 512
LOSS_SCALE = 1.25 / (N * D)


def _argmin_body(z_ref, e_ref, idx_ref, loss_ref):
    i = pl.program_id(0)
    z = z_ref[...]
    z2 = jnp.sum(z * z, axis=1, keepdims=True)

    def step(kk, carry):
        run_min, run_idx = carry
        e = e_ref[pl.ds(kk * BK, BK), :]
        e2 = jnp.sum(e * e, axis=1)[None, :]
        dot = lax.dot_general(z, e, (((1,), (1,)), ((), ())),
                              preferred_element_type=jnp.float32)
        d2 = jnp.maximum((z2 + e2) - 2.0 * dot, 0.0)
        dist = jnp.sqrt(d2)
        mn = jnp.min(dist, axis=1, keepdims=True)
        ids = lax.broadcasted_iota(jnp.int32, (BN, BK), 1) + kk * BK
        am = jnp.min(jnp.where(dist == mn, ids, K), axis=1, keepdims=True)
        better = mn < run_min
        return (jnp.where(better, mn, run_min),
                jnp.where(better, am, run_idx))

    init = (jnp.full((BN, 1), jnp.inf, jnp.float32),
            jnp.zeros((BN, 1), jnp.int32))
    run_min, run_idx = lax.fori_loop(0, K // BK, step, init)
    idx_ref[...] = run_idx[:, 0]
    part = (jnp.sum(run_min * run_min) * LOSS_SCALE).reshape(1, 1)

    @pl.when(i == 0)
    def _():
        loss_ref[...] = part

    @pl.when(i > 0)
    def _():
        loss_ref[...] = loss_ref[...] + part


_argmin_call = pl.pallas_call(
    _argmin_body,
    grid=(N // BN,),
    in_specs=[
        pl.BlockSpec((BN, D), lambda i: (i, 0)),
        pl.BlockSpec((K, D), lambda i: (0, 0)),
    ],
    out_specs=[
        pl.BlockSpec((BN,), lambda i: (i,)),
        pl.BlockSpec((1, 1), lambda i: (0, 0)),
    ],
    out_shape=[
        jax.ShapeDtypeStruct((N,), jnp.int32),
        jax.ShapeDtypeStruct((1, 1), jnp.float32),
    ],
)

_NW = 32
_BPW = N // _NW
_CH = 128


@functools.cache
def _sc_gather():
    @functools.partial(
        pl.kernel,
        mesh=plsc.VectorSubcoreMesh(core_axis_name="c", subcore_axis_name="s"),
        out_type=jax.ShapeDtypeStruct((N, D), jnp.float32),
        scratch_types=[
            pltpu.VMEM((_CH,), jnp.int32),
            pltpu.VMEM((_CH, D), jnp.float32),
            pltpu.SemaphoreType.DMA,
        ],
    )
    def gather(table_hbm, idx_hbm, out_hbm, idx_v, rows_v, sem):
        wid = lax.axis_index("s") * 2 + lax.axis_index("c")
        base = wid * _BPW
        for c in range(_BPW // _CH):
            o = base + c * _CH
            pltpu.sync_copy(idx_hbm.at[pl.ds(o, _CH)], idx_v)
            pltpu.async_copy(table_hbm.at[idx_v], rows_v, sem).wait()
            pltpu.sync_copy(rows_v, out_hbm.at[pl.ds(o, _CH)])

    return gather


def kernel(z, embeddings):
    idx, loss = _argmin_call(z, embeddings)
    quantized = _sc_gather()(embeddings, idx)
    return idx, quantized, loss[0, 0]
